# Initial kernel scaffold; baseline (speedup 1.0000x reference)
import functools

import jax
import jax.numpy as jnp
from jax import lax
from jax.experimental import pallas as pl
from jax.experimental.pallas import tpu as pltpu
from jax.experimental.pallas import tpu_sc as plsc

B = 16
S = 512
D = 768
NCAND = 32
TPAD = B * S
BLK = 256
NBLK = TPAD // BLK
NW = 32
CW_MAX = TPAD // NW

_sc_mesh = plsc.VectorSubcoreMesh(core_axis_name="c", subcore_axis_name="s")


@functools.partial(
    pl.kernel,
    mesh=_sc_mesh,
    out_type=(
        jax.ShapeDtypeStruct((TPAD, D), jnp.float32),
        jax.ShapeDtypeStruct((B * NCAND, D), jnp.float32),
        jax.ShapeDtypeStruct((B * NCAND,), jnp.float32),
    ),
    scratch_types=[
        pltpu.VMEM((B * S,), jnp.int32),
        pltpu.VMEM((CW_MAX,), jnp.int32),
        pltpu.VMEM((16,), jnp.int32),
        pltpu.VMEM((16, D), jnp.float32),
        pltpu.VMEM((16,), jnp.int32),
        pltpu.VMEM((16, D), jnp.float32),
        pltpu.VMEM((16,), jnp.float32),
        pltpu.SemaphoreType.DMA,
        pltpu.SemaphoreType.DMA,
    ],
)
def _sc_gather(ctx_hbm, pos_hbm, meta_hbm, sids_hbm, emb_hbm, sw_hbm, sb_hbm,
               tok_out, wg_out, bg_out,
               ctx_v, pos_v, meta_v, rows_v, sidx_v, srows_v, sb_v, sem0, sem1):
    wid = lax.axis_index("s") * 2 + lax.axis_index("c")

    pltpu.sync_copy(sids_hbm.at[pl.ds(wid * 16, 16)], sidx_v)
    pltpu.async_copy(sw_hbm.at[sidx_v], srows_v, sem0).wait()
    pltpu.sync_copy(srows_v, wg_out.at[pl.ds(wid * 16, 16)])
    pltpu.async_copy(sb_hbm.at[sidx_v], sb_v, sem0).wait()
    pltpu.sync_copy(sb_v, bg_out.at[pl.ds(wid * 16, 16)])

    pltpu.sync_copy(meta_hbm, meta_v)
    cw = jnp.max(meta_v[...])
    nch = cw // 16
    base = wid * cw
    pltpu.sync_copy(ctx_hbm, ctx_v)
    pltpu.sync_copy(pos_hbm.at[pl.ds(base, CW_MAX)], pos_v)

    def body(j, carry):
        pos16 = pos_v[pl.ds(j * 16, 16)]
        ids16 = plsc.load_gather(ctx_v, [pos16])
        pltpu.async_copy(emb_hbm.at[ids16], rows_v, sem1).wait()
        pltpu.sync_copy(rows_v, tok_out.at[pl.ds(base + j * 16, 16)])
        return carry

    lax.fori_loop(0, nch, body, 0)


def _tc_body(nb_ref, tok_ref, w_ref, b_ref, lo_ref, hi_ref, iw_ref,
             wg_ref, bgr_ref, tgt_ref, loss_ref, corr_ref, acc_ref):
    i = pl.program_id(0)
    nb = nb_ref[0]

    @pl.when(i == 0)
    def _():
        acc_ref[...] = jnp.zeros_like(acc_ref)

    @pl.when(i < nb)
    def _():
        h = jnp.tanh(
            jnp.dot(tok_ref[...], w_ref[...], preferred_element_type=jnp.float32)
            + b_ref[...])
        gcol = i * BLK + lax.broadcasted_iota(jnp.int32, (B, BLK), 1)
        m = ((gcol >= lo_ref[...]) & (gcol < hi_ref[...])).astype(jnp.float32)
        m = m * iw_ref[...]
        acc_ref[...] += jnp.dot(m, h, preferred_element_type=jnp.float32)

    @pl.when(i == NBLK - 1)
    def _():
        reps = acc_ref[...]
        rows = []
        for b in range(B):
            wgb = wg_ref[pl.ds(b * NCAND, NCAND), :]
            rb = reps[b:b + 1, :]
            rows.append(lax.dot_general(
                rb, wgb, (((1,), (1,)), ((), ())),
                preferred_element_type=jnp.float32))
        logits = jnp.concatenate(rows, axis=0) + bgr_ref[...]

        mx = jnp.max(logits, axis=1, keepdims=True)
        ex = jnp.exp(logits - mx)
        z = jnp.sum(ex, axis=1, keepdims=True)
        logz = jnp.log(z) + mx
        ci = lax.broadcasted_iota(jnp.int32, (B, NCAND), 1)
        tgt = tgt_ref[...]
        tl = jnp.sum(jnp.where(ci == tgt, logits, 0.0), axis=1, keepdims=True)
        loss_ref[...] = jnp.sum((logz - tl) * (1.0 / B), axis=0, keepdims=True)
        amax = jnp.min(jnp.where(logits == mx, ci, NCAND), axis=1, keepdims=True)
        corr_ref[...] = (amax == tgt).astype(jnp.int32)


def _tc_forward(nb_arr, tok, w_enc, b_enc2, lo, hi, iw, wg, bgr, tgt2):
    grid_spec = pltpu.PrefetchScalarGridSpec(
        num_scalar_prefetch=1,
        grid=(NBLK,),
        in_specs=[
            pl.BlockSpec((BLK, D), lambda i, nb: (jnp.minimum(i, nb[0] - 1), 0)),
            pl.BlockSpec((D, D), lambda i, nb: (0, 0)),
            pl.BlockSpec((1, D), lambda i, nb: (0, 0)),
            pl.BlockSpec((B, 1), lambda i, nb: (0, 0)),
            pl.BlockSpec((B, 1), lambda i, nb: (0, 0)),
            pl.BlockSpec((B, 1), lambda i, nb: (0, 0)),
            pl.BlockSpec((B * NCAND, D), lambda i, nb: (0, 0)),
            pl.BlockSpec((B, NCAND), lambda i, nb: (0, 0)),
            pl.BlockSpec((B, 1), lambda i, nb: (0, 0)),
        ],
        out_specs=[
            pl.BlockSpec((1, 1), lambda i, nb: (0, 0)),
            pl.BlockSpec((B, 1), lambda i, nb: (0, 0)),
        ],
        scratch_shapes=[pltpu.VMEM((B, D), jnp.float32)],
    )
    return pl.pallas_call(
        _tc_body,
        grid_spec=grid_spec,
        out_shape=[
            jax.ShapeDtypeStruct((1, 1), jnp.float32),
            jax.ShapeDtypeStruct((B, 1), jnp.int32),
        ],
    )(nb_arr, tok, w_enc, b_enc2, lo, hi, iw, wg, bgr, tgt2)


def kernel(context_ids, context_spans, sense_ids, target_ids, emb_table,
           W_enc, b_enc, sense_W, sense_b):
    context_ids = context_ids.astype(jnp.int32)
    context_spans = context_spans.astype(jnp.int32)
    sense_ids = sense_ids.astype(jnp.int32)
    target_ids = target_ids.astype(jnp.int32)

    start = context_spans[:, 0]
    end = jnp.maximum(context_spans[:, 1], start + 1)
    w = (end - start).astype(jnp.int32)
    cum = jnp.concatenate([jnp.zeros((1,), jnp.int32),
                           jnp.cumsum(w, dtype=jnp.int32)])
    t_total = cum[B]
    tp = ((t_total + 511) // 512) * 512
    cw = tp // NW
    nb = tp // BLK

    t = jnp.arange(TPAD, dtype=jnp.int32)
    b_of_t = jnp.minimum(
        jnp.searchsorted(cum[1:], t, side="right").astype(jnp.int32), B - 1)
    pos = start[b_of_t] + (t - cum[b_of_t]) + b_of_t * S
    pos = jnp.where(t < t_total, pos, 0)

    meta = jnp.full((16,), cw, dtype=jnp.int32)
    ctx_flat = context_ids.reshape(-1)
    sids_flat = sense_ids.reshape(-1)

    tok, wg, bg = _sc_gather(ctx_flat, pos, meta, sids_flat,
                             emb_table, sense_W, sense_b)

    nb_arr = jnp.reshape(nb, (1,)).astype(jnp.int32)
    lo = cum[:B].reshape(B, 1)
    hi = cum[1:].reshape(B, 1)
    iw = (1.0 / w.astype(jnp.float32)).reshape(B, 1)
    loss2, corr2 = _tc_forward(nb_arr, tok, W_enc, b_enc.reshape(1, D),
                               lo, hi, iw, wg, bg.reshape(B, NCAND),
                               target_ids.reshape(B, 1))
    return loss2[0, 0], corr2[:, 0].astype(jnp.bool_)


# speedup vs baseline: 1.3423x; 1.3423x over previous
import functools

import jax
import jax.numpy as jnp
from jax import lax
from jax.experimental import pallas as pl
from jax.experimental.pallas import tpu as pltpu
from jax.experimental.pallas import tpu_sc as plsc

B = 16
S = 512
D = 768
NCAND = 32
TPAD = B * S
BLK = 256
NBLK = TPAD // BLK
NW = 32
CW_MAX = TPAD // NW

@functools.lru_cache(maxsize=None)
def _make_sc_gather():
    mesh = plsc.VectorSubcoreMesh(core_axis_name="c", subcore_axis_name="s")

    @functools.partial(
        pl.kernel,
        mesh=mesh,
        compiler_params=pltpu.CompilerParams(needs_layout_passes=False),
        out_type=(
            jax.ShapeDtypeStruct((TPAD, D), jnp.float32),
            jax.ShapeDtypeStruct((B * NCAND, D), jnp.float32),
            jax.ShapeDtypeStruct((B * NCAND,), jnp.float32),
        ),
        scratch_types=[
            pltpu.VMEM((B * S,), jnp.int32),
            pltpu.VMEM((CW_MAX,), jnp.int32),
            pltpu.VMEM((16,), jnp.int32),
            pltpu.VMEM((16,), jnp.int32),
            pltpu.VMEM((16, D), jnp.float32),
            pltpu.VMEM((16,), jnp.int32),
            pltpu.VMEM((16,), jnp.int32),
            pltpu.VMEM((16, D), jnp.float32),
            pltpu.VMEM((16, 128), jnp.float32),
            pltpu.VMEM((16,), jnp.float32),
            pltpu.SemaphoreType.DMA,
            pltpu.SemaphoreType.DMA,
        ],
    )
    def sc_gather(ctx_hbm, pos_hbm, meta_hbm, sids_hbm, emb_hbm, sw_hbm, sb_hbm,
                  tok_out, wg_out, bg_out,
                  ctx_v, pos_v, meta_v, ids_v, rows_v, sidx_v, sidx_hi_v,
                  srows_v, sbrows_v, sb_v, sem0, sem1):
        wid = lax.axis_index("s") * 2 + lax.axis_index("c")
        sbase = pl.multiple_of(wid * 16, 16)

        pltpu.sync_copy(sids_hbm.at[pl.ds(sbase, 16)], sidx_v)
        pltpu.async_copy(sw_hbm.at[sidx_v], srows_v, sem0).wait()
        pltpu.sync_copy(srows_v, wg_out.at[pl.ds(sbase, 16)])
        sids = sidx_v[...]
        sidx_hi_v[...] = jnp.right_shift(sids, 7)
        pltpu.async_copy(sb_hbm.at[sidx_hi_v], sbrows_v, sem0).wait()
        lane = lax.iota(jnp.int32, 16)
        sb_v[...] = plsc.load_gather(sbrows_v, [lane, jnp.bitwise_and(sids, 127)])
        pltpu.sync_copy(sb_v, bg_out.at[pl.ds(sbase, 16)])

        pltpu.sync_copy(meta_hbm, meta_v)
        cw = jnp.max(meta_v[...])
        nch = cw // 16
        base = pl.multiple_of(wid * cw, 16)
        pltpu.sync_copy(ctx_hbm, ctx_v)
        pltpu.sync_copy(pos_hbm.at[pl.ds(base, CW_MAX)], pos_v)

        def body(j, carry):
            pos16 = pos_v[pl.ds(pl.multiple_of(j * 16, 16), 16)]
            ids_v[...] = plsc.load_gather(ctx_v, [pos16])
            pltpu.async_copy(emb_hbm.at[ids_v], rows_v, sem1).wait()
            pltpu.sync_copy(rows_v, tok_out.at[pl.ds(pl.multiple_of(base + j * 16, 16), 16)])
            return carry

        lax.fori_loop(0, nch, body, 0)

    return sc_gather


def _sc_gather(*args):
    return _make_sc_gather()(*args)


def _tc_body(nb_ref, tok_ref, w_ref, b_ref, lo_ref, hi_ref, iw_ref,
             wg_ref, bgr_ref, tgt_ref, loss_ref, corr_ref, acc_ref):
    i = pl.program_id(0)
    nb = nb_ref[0]

    @pl.when(i == 0)
    def _():
        acc_ref[...] = jnp.zeros_like(acc_ref)

    @pl.when(i < nb)
    def _():
        h = jnp.tanh(
            jnp.dot(tok_ref[...], w_ref[...], preferred_element_type=jnp.float32)
            + b_ref[...])
        gcol = i * BLK + lax.broadcasted_iota(jnp.int32, (B, BLK), 1)
        m = ((gcol >= lo_ref[...]) & (gcol < hi_ref[...])).astype(jnp.float32)
        m = m * iw_ref[...]
        acc_ref[...] += jnp.dot(m, h, preferred_element_type=jnp.float32)

    @pl.when(i == NBLK - 1)
    def _():
        reps = acc_ref[...]
        rows = []
        for b in range(B):
            wgb = wg_ref[pl.ds(b * NCAND, NCAND), :]
            rb = reps[b:b + 1, :]
            rows.append(lax.dot_general(
                rb, wgb, (((1,), (1,)), ((), ())),
                preferred_element_type=jnp.float32))
        logits = jnp.concatenate(rows, axis=0) + bgr_ref[...]

        mx = jnp.max(logits, axis=1, keepdims=True)
        ex = jnp.exp(logits - mx)
        z = jnp.sum(ex, axis=1, keepdims=True)
        logz = jnp.log(z) + mx
        ci = lax.broadcasted_iota(jnp.int32, (B, NCAND), 1)
        tgt = tgt_ref[...]
        tl = jnp.sum(jnp.where(ci == tgt, logits, 0.0), axis=1, keepdims=True)
        loss_ref[...] = jnp.sum((logz - tl) * (1.0 / B), axis=0, keepdims=True)
        amax = jnp.min(jnp.where(logits == mx, ci, NCAND), axis=1, keepdims=True)
        corr_ref[...] = (amax == tgt).astype(jnp.int32)


def _tc_forward(nb_arr, tok, w_enc, b_enc2, lo, hi, iw, wg, bgr, tgt2):
    grid_spec = pltpu.PrefetchScalarGridSpec(
        num_scalar_prefetch=1,
        grid=(NBLK,),
        in_specs=[
            pl.BlockSpec((BLK, D), lambda i, nb: (jnp.minimum(i, nb[0] - 1), 0)),
            pl.BlockSpec((D, D), lambda i, nb: (0, 0)),
            pl.BlockSpec((1, D), lambda i, nb: (0, 0)),
            pl.BlockSpec((B, 1), lambda i, nb: (0, 0)),
            pl.BlockSpec((B, 1), lambda i, nb: (0, 0)),
            pl.BlockSpec((B, 1), lambda i, nb: (0, 0)),
            pl.BlockSpec((B * NCAND, D), lambda i, nb: (0, 0)),
            pl.BlockSpec((B, NCAND), lambda i, nb: (0, 0)),
            pl.BlockSpec((B, 1), lambda i, nb: (0, 0)),
        ],
        out_specs=[
            pl.BlockSpec((1, 1), lambda i, nb: (0, 0)),
            pl.BlockSpec((B, 1), lambda i, nb: (0, 0)),
        ],
        scratch_shapes=[pltpu.VMEM((B, D), jnp.float32)],
    )
    return pl.pallas_call(
        _tc_body,
        grid_spec=grid_spec,
        out_shape=[
            jax.ShapeDtypeStruct((1, 1), jnp.float32),
            jax.ShapeDtypeStruct((B, 1), jnp.int32),
        ],
    )(nb_arr, tok, w_enc, b_enc2, lo, hi, iw, wg, bgr, tgt2)


def kernel(context_ids, context_spans, sense_ids, target_ids, emb_table,
           W_enc, b_enc, sense_W, sense_b):
    context_ids = context_ids.astype(jnp.int32)
    context_spans = context_spans.astype(jnp.int32)
    sense_ids = sense_ids.astype(jnp.int32)
    target_ids = target_ids.astype(jnp.int32)

    start = context_spans[:, 0]
    end = jnp.maximum(context_spans[:, 1], start + 1)
    w = (end - start).astype(jnp.int32)
    cum = jnp.concatenate([jnp.zeros((1,), jnp.int32),
                           jnp.cumsum(w, dtype=jnp.int32)])
    t_total = cum[B]
    tp = ((t_total + 511) // 512) * 512
    cw = tp // NW
    nb = tp // BLK

    t = jnp.arange(TPAD, dtype=jnp.int32)
    b_of_t = jnp.minimum(
        jnp.searchsorted(cum[1:], t, side="right").astype(jnp.int32), B - 1)
    pos = start[b_of_t] + (t - cum[b_of_t]) + b_of_t * S
    pos = jnp.where(t < t_total, pos, 0)

    meta = jnp.full((16,), cw, dtype=jnp.int32)
    ctx_flat = context_ids.reshape(-1)
    sids_flat = sense_ids.reshape(-1)

    n_senses = sense_b.shape[0]
    pad_b = (-n_senses) % 128
    sb_rows = jnp.pad(sense_b, (0, pad_b)).reshape(-1, 128)
    tok, wg, bg = _sc_gather(ctx_flat, pos, meta, sids_flat,
                             emb_table, sense_W, sb_rows)

    nb_arr = jnp.reshape(nb, (1,)).astype(jnp.int32)
    lo = cum[:B].reshape(B, 1)
    hi = cum[1:].reshape(B, 1)
    iw = (1.0 / w.astype(jnp.float32)).reshape(B, 1)
    loss2, corr2 = _tc_forward(nb_arr, tok, W_enc, b_enc.reshape(1, D),
                               lo, hi, iw, wg, bg.reshape(B, NCAND),
                               target_ids.reshape(B, 1))
    return loss2[0, 0], corr2[:, 0].astype(jnp.bool_)


# speedup vs baseline: 2.7760x; 2.0680x over previous
import functools

import jax
import jax.numpy as jnp
from jax import lax
from jax.experimental import pallas as pl
from jax.experimental.pallas import tpu as pltpu
from jax.experimental.pallas import tpu_sc as plsc

B = 16
S = 512
D = 768
NCAND = 32
TPAD = B * S
BLK = 256
NBLK = TPAD // BLK
NW = 32
CW_MAX = TPAD // NW

@functools.lru_cache(maxsize=None)
def _make_sc_gather():
    mesh = plsc.VectorSubcoreMesh(core_axis_name="c", subcore_axis_name="s")

    @functools.partial(
        pl.kernel,
        mesh=mesh,
        compiler_params=pltpu.CompilerParams(needs_layout_passes=False),
        out_type=(
            jax.ShapeDtypeStruct((TPAD, D), jnp.float32),
            jax.ShapeDtypeStruct((B * NCAND, D), jnp.float32),
            jax.ShapeDtypeStruct((B * NCAND,), jnp.float32),
        ),
        scratch_types=[
            pltpu.VMEM((B * S,), jnp.int32),
            pltpu.VMEM((CW_MAX,), jnp.int32),
            pltpu.VMEM((16,), jnp.int32),
            pltpu.VMEM((16,), jnp.int32),
            pltpu.VMEM((16, D), jnp.float32),
            pltpu.VMEM((16,), jnp.int32),
            pltpu.VMEM((16,), jnp.int32),
            pltpu.VMEM((16, D), jnp.float32),
            pltpu.VMEM((16, 128), jnp.float32),
            pltpu.VMEM((16,), jnp.float32),
            pltpu.SemaphoreType.DMA,
            pltpu.SemaphoreType.DMA,
        ],
    )
    def sc_gather(ctx_hbm, pos_hbm, meta_hbm, sids_hbm, emb_hbm, sw_hbm, sb_hbm,
                  tok_out, wg_out, bg_out,
                  ctx_v, pos_v, meta_v, ids_v, rows_v, sidx_v, sidx_hi_v,
                  srows_v, sbrows_v, sb_v, sem0, sem1):
        wid = lax.axis_index("s") * 2 + lax.axis_index("c")
        sbase = pl.multiple_of(wid * 16, 16)

        pltpu.sync_copy(sids_hbm.at[pl.ds(sbase, 16)], sidx_v)
        pltpu.async_copy(sw_hbm.at[sidx_v], srows_v, sem0).wait()
        pltpu.sync_copy(srows_v, wg_out.at[pl.ds(sbase, 16)])
        sids = sidx_v[...]
        sidx_hi_v[...] = jnp.right_shift(sids, 7)
        pltpu.async_copy(sb_hbm.at[sidx_hi_v], sbrows_v, sem0).wait()
        lane = lax.iota(jnp.int32, 16)
        sb_v[...] = plsc.load_gather(sbrows_v, [lane, jnp.bitwise_and(sids, 127)])
        pltpu.sync_copy(sb_v, bg_out.at[pl.ds(sbase, 16)])

        pltpu.sync_copy(meta_hbm, meta_v)
        cw = jnp.max(meta_v[...])
        nch = cw // 16
        base = pl.multiple_of(wid * cw, 16)
        pltpu.sync_copy(ctx_hbm, ctx_v)
        pltpu.sync_copy(pos_hbm.at[pl.ds(base, CW_MAX)], pos_v)

        def body(j, carry):
            pos16 = pos_v[pl.ds(pl.multiple_of(j * 16, 16), 16)]
            ids_v[...] = plsc.load_gather(ctx_v, [pos16])
            pltpu.async_copy(emb_hbm.at[ids_v], rows_v, sem1).wait()
            pltpu.sync_copy(rows_v, tok_out.at[pl.ds(pl.multiple_of(base + j * 16, 16), 16)])
            return carry

        lax.fori_loop(0, nch, body, 0)

    return sc_gather


def _sc_gather(*args):
    return _make_sc_gather()(*args)


def _tc_body(nb_ref, tok_ref, w_ref, b_ref, lo_ref, hi_ref, iw_ref,
             wg_ref, bgr_ref, tgt_ref, loss_ref, corr_ref, acc_ref):
    i = pl.program_id(0)
    nb = nb_ref[0]

    @pl.when(i == 0)
    def _():
        acc_ref[...] = jnp.zeros_like(acc_ref)

    @pl.when(i < nb)
    def _():
        h = jnp.tanh(
            jnp.dot(tok_ref[...], w_ref[...], preferred_element_type=jnp.float32)
            + b_ref[...])
        gcol = i * BLK + lax.broadcasted_iota(jnp.int32, (B, BLK), 1)
        m = ((gcol >= lo_ref[...]) & (gcol < hi_ref[...])).astype(jnp.float32)
        m = m * iw_ref[...]
        acc_ref[...] += jnp.dot(m, h, preferred_element_type=jnp.float32)

    @pl.when(i == NBLK - 1)
    def _():
        reps = acc_ref[...]
        rows = []
        for b in range(B):
            wgb = wg_ref[pl.ds(b * NCAND, NCAND), :]
            rb = reps[b:b + 1, :]
            rows.append(lax.dot_general(
                rb, wgb, (((1,), (1,)), ((), ())),
                preferred_element_type=jnp.float32))
        logits = jnp.concatenate(rows, axis=0) + bgr_ref[...]

        mx = jnp.max(logits, axis=1, keepdims=True)
        ex = jnp.exp(logits - mx)
        z = jnp.sum(ex, axis=1, keepdims=True)
        logz = jnp.log(z) + mx
        ci = lax.broadcasted_iota(jnp.int32, (B, NCAND), 1)
        tgt = tgt_ref[...]
        tl = jnp.sum(jnp.where(ci == tgt, logits, 0.0), axis=1, keepdims=True)
        loss_ref[...] = jnp.sum((logz - tl) * (1.0 / B), axis=0, keepdims=True)
        amax = jnp.min(jnp.where(logits == mx, ci, NCAND), axis=1, keepdims=True)
        corr_ref[...] = (amax == tgt).astype(jnp.int32)


def _tc_forward(nb_arr, tok, w_enc, b_enc2, lo, hi, iw, wg, bgr, tgt2):
    grid_spec = pltpu.PrefetchScalarGridSpec(
        num_scalar_prefetch=1,
        grid=(NBLK,),
        in_specs=[
            pl.BlockSpec((BLK, D), lambda i, nb: (jnp.minimum(i, nb[0] - 1), 0)),
            pl.BlockSpec((D, D), lambda i, nb: (0, 0)),
            pl.BlockSpec((1, D), lambda i, nb: (0, 0)),
            pl.BlockSpec((B, 1), lambda i, nb: (0, 0)),
            pl.BlockSpec((B, 1), lambda i, nb: (0, 0)),
            pl.BlockSpec((B, 1), lambda i, nb: (0, 0)),
            pl.BlockSpec((B * NCAND, D), lambda i, nb: (0, 0)),
            pl.BlockSpec((B, NCAND), lambda i, nb: (0, 0)),
            pl.BlockSpec((B, 1), lambda i, nb: (0, 0)),
        ],
        out_specs=[
            pl.BlockSpec((1, 1), lambda i, nb: (0, 0)),
            pl.BlockSpec((B, 1), lambda i, nb: (0, 0)),
        ],
        scratch_shapes=[pltpu.VMEM((B, D), jnp.float32)],
    )
    return pl.pallas_call(
        _tc_body,
        grid_spec=grid_spec,
        out_shape=[
            jax.ShapeDtypeStruct((1, 1), jnp.float32),
            jax.ShapeDtypeStruct((B, 1), jnp.int32),
        ],
    )(nb_arr, tok, w_enc, b_enc2, lo, hi, iw, wg, bgr, tgt2)


def kernel(context_ids, context_spans, sense_ids, target_ids, emb_table,
           W_enc, b_enc, sense_W, sense_b):
    context_ids = context_ids.astype(jnp.int32)
    context_spans = context_spans.astype(jnp.int32)
    sense_ids = sense_ids.astype(jnp.int32)
    target_ids = target_ids.astype(jnp.int32)

    start = context_spans[:, 0]
    end = jnp.maximum(context_spans[:, 1], start + 1)
    w = (end - start).astype(jnp.int32)
    cum = jnp.concatenate([jnp.zeros((1,), jnp.int32),
                           jnp.cumsum(w, dtype=jnp.int32)])
    t_total = cum[B]
    tp = ((t_total + 511) // 512) * 512
    cw = tp // NW
    nb = tp // BLK

    t = jnp.arange(TPAD, dtype=jnp.int32)
    ge_hi = (t[None, :] >= cum[1:, None]).astype(jnp.int32)
    b_of_t = jnp.minimum(jnp.sum(ge_hi, axis=0), B - 1)
    onehot = (b_of_t[None, :] == jnp.arange(B, dtype=jnp.int32)[:, None])
    start_sel = jnp.sum(jnp.where(onehot, start[:, None], 0), axis=0)
    cum_sel = jnp.sum(jnp.where(onehot, cum[:B, None], 0), axis=0)
    pos = start_sel + (t - cum_sel) + b_of_t * S
    pos = jnp.where(t < t_total, pos, 0)

    meta = jnp.full((16,), cw, dtype=jnp.int32)
    ctx_flat = context_ids.reshape(-1)
    sids_flat = sense_ids.reshape(-1)

    n_senses = sense_b.shape[0]
    pad_b = (-n_senses) % 128
    sb_rows = jnp.pad(sense_b, (0, pad_b)).reshape(-1, 128)
    tok, wg, bg = _sc_gather(ctx_flat, pos, meta, sids_flat,
                             emb_table, sense_W, sb_rows)

    nb_arr = jnp.reshape(nb, (1,)).astype(jnp.int32)
    lo = cum[:B].reshape(B, 1)
    hi = cum[1:].reshape(B, 1)
    iw = (1.0 / w.astype(jnp.float32)).reshape(B, 1)
    loss2, corr2 = _tc_forward(nb_arr, tok, W_enc, b_enc.reshape(1, D),
                               lo, hi, iw, wg, bg.reshape(B, NCAND),
                               target_ids.reshape(B, 1))
    return loss2[0, 0], corr2[:, 0].astype(jnp.bool_)


# speedup vs baseline: 2.9743x; 1.0714x over previous
import functools

import jax
import jax.numpy as jnp
from jax import lax
from jax.experimental import pallas as pl
from jax.experimental.pallas import tpu as pltpu
from jax.experimental.pallas import tpu_sc as plsc

B = 16
S = 512
D = 768
NCAND = 32
TPAD = B * S
BLK = 512
NBLK = TPAD // BLK
NW = 32
CW_MAX = TPAD // NW

@functools.lru_cache(maxsize=None)
def _make_sc_gather():
    mesh = plsc.VectorSubcoreMesh(core_axis_name="c", subcore_axis_name="s")

    @functools.partial(
        pl.kernel,
        mesh=mesh,
        compiler_params=pltpu.CompilerParams(needs_layout_passes=False),
        out_type=(
            jax.ShapeDtypeStruct((TPAD, D), jnp.float32),
            jax.ShapeDtypeStruct((B * NCAND, D), jnp.float32),
            jax.ShapeDtypeStruct((B * NCAND,), jnp.float32),
        ),
        scratch_types=[
            pltpu.VMEM((B * S,), jnp.int32),
            pltpu.VMEM((CW_MAX,), jnp.int32),
            pltpu.VMEM((16,), jnp.int32),
            pltpu.VMEM((64,), jnp.int32),
            pltpu.VMEM((64, D), jnp.float32),
            pltpu.VMEM((16,), jnp.int32),
            pltpu.VMEM((16, D), jnp.float32),
            pltpu.VMEM((16,), jnp.int32),
            pltpu.VMEM((16,), jnp.int32),
            pltpu.VMEM((16, D), jnp.float32),
            pltpu.VMEM((16, 128), jnp.float32),
            pltpu.VMEM((16,), jnp.float32),
            pltpu.SemaphoreType.DMA,
            pltpu.SemaphoreType.DMA,
        ],
    )
    def sc_gather(ctx_hbm, pos_hbm, meta_hbm, sids_hbm, emb_hbm, sw_hbm, sb_hbm,
                  tok_out, wg_out, bg_out,
                  ctx_v, pos_v, meta_v, ids64_v, rows64_v, ids_v, rows_v,
                  sidx_v, sidx_hi_v, srows_v, sbrows_v, sb_v, sem0, sem1):
        wid = lax.axis_index("s") * 2 + lax.axis_index("c")
        sbase = pl.multiple_of(wid * 16, 16)

        pltpu.sync_copy(sids_hbm.at[pl.ds(sbase, 16)], sidx_v)
        pltpu.async_copy(sw_hbm.at[sidx_v], srows_v, sem0).wait()
        pltpu.sync_copy(srows_v, wg_out.at[pl.ds(sbase, 16)])
        sids = sidx_v[...]
        sidx_hi_v[...] = jnp.right_shift(sids, 7)
        pltpu.async_copy(sb_hbm.at[sidx_hi_v], sbrows_v, sem0).wait()
        lane = lax.iota(jnp.int32, 16)
        sb_v[...] = plsc.load_gather(sbrows_v, [lane, jnp.bitwise_and(sids, 127)])
        pltpu.sync_copy(sb_v, bg_out.at[pl.ds(sbase, 16)])

        pltpu.sync_copy(meta_hbm, meta_v)
        cw = jnp.max(meta_v[...])
        nch = cw // 16
        base = pl.multiple_of(wid * cw, 16)
        pltpu.sync_copy(ctx_hbm, ctx_v)
        pltpu.sync_copy(pos_hbm.at[pl.ds(base, CW_MAX)], pos_v)

        nch64 = cw // 64
        rem = (cw - nch64 * 64) // 16

        def body64(k, carry):
            for i in range(4):
                off = pl.multiple_of(k * 64 + i * 16, 16)
                ids64_v[pl.ds(i * 16, 16)] = plsc.load_gather(
                    ctx_v, [pos_v[pl.ds(off, 16)]])
            pltpu.async_copy(emb_hbm.at[ids64_v], rows64_v, sem1).wait()
            pltpu.sync_copy(
                rows64_v, tok_out.at[pl.ds(pl.multiple_of(base + k * 64, 16), 64)])
            return carry

        lax.fori_loop(0, nch64, body64, 0)
        j0 = nch64 * 64

        def body16(j, carry):
            off = pl.multiple_of(j0 + j * 16, 16)
            ids_v[...] = plsc.load_gather(ctx_v, [pos_v[pl.ds(off, 16)]])
            pltpu.async_copy(emb_hbm.at[ids_v], rows_v, sem1).wait()
            pltpu.sync_copy(
                rows_v, tok_out.at[pl.ds(pl.multiple_of(base + j0 + j * 16, 16), 16)])
            return carry

        lax.fori_loop(0, rem, body16, 0)

    return sc_gather


def _sc_gather(*args):
    return _make_sc_gather()(*args)


def _tc_body(nb_ref, tok_ref, w_ref, b_ref, lo_ref, hi_ref, iw_ref,
             wg_ref, bgr_ref, tgt_ref, loss_ref, corr_ref, acc_ref):
    i = pl.program_id(0)
    nb = nb_ref[0]

    @pl.when(i == 0)
    def _():
        acc_ref[...] = jnp.zeros_like(acc_ref)

    @pl.when(i < nb)
    def _():
        h = jnp.tanh(
            jnp.dot(tok_ref[...], w_ref[...], preferred_element_type=jnp.float32)
            + b_ref[...])
        gcol = i * BLK + lax.broadcasted_iota(jnp.int32, (B, BLK), 1)
        m = ((gcol >= lo_ref[...]) & (gcol < hi_ref[...])).astype(jnp.float32)
        m = m * iw_ref[...]
        acc_ref[...] += jnp.dot(m, h, preferred_element_type=jnp.float32)

    @pl.when(i == NBLK - 1)
    def _():
        reps = acc_ref[...]
        rows = []
        for b in range(B):
            wgb = wg_ref[pl.ds(b * NCAND, NCAND), :]
            rb = reps[b:b + 1, :]
            rows.append(lax.dot_general(
                rb, wgb, (((1,), (1,)), ((), ())),
                preferred_element_type=jnp.float32))
        logits = jnp.concatenate(rows, axis=0) + bgr_ref[...]

        mx = jnp.max(logits, axis=1, keepdims=True)
        ex = jnp.exp(logits - mx)
        z = jnp.sum(ex, axis=1, keepdims=True)
        logz = jnp.log(z) + mx
        ci = lax.broadcasted_iota(jnp.int32, (B, NCAND), 1)
        tgt = tgt_ref[...]
        tl = jnp.sum(jnp.where(ci == tgt, logits, 0.0), axis=1, keepdims=True)
        loss_ref[...] = jnp.sum((logz - tl) * (1.0 / B), axis=0, keepdims=True)
        amax = jnp.min(jnp.where(logits == mx, ci, NCAND), axis=1, keepdims=True)
        corr_ref[...] = (amax == tgt).astype(jnp.int32)


def _tc_forward(nb_arr, tok, w_enc, b_enc2, lo, hi, iw, wg, bgr, tgt2):
    grid_spec = pltpu.PrefetchScalarGridSpec(
        num_scalar_prefetch=1,
        grid=(NBLK,),
        in_specs=[
            pl.BlockSpec((BLK, D), lambda i, nb: (jnp.minimum(i, nb[0] - 1), 0)),
            pl.BlockSpec((D, D), lambda i, nb: (0, 0)),
            pl.BlockSpec((1, D), lambda i, nb: (0, 0)),
            pl.BlockSpec((B, 1), lambda i, nb: (0, 0)),
            pl.BlockSpec((B, 1), lambda i, nb: (0, 0)),
            pl.BlockSpec((B, 1), lambda i, nb: (0, 0)),
            pl.BlockSpec((B * NCAND, D), lambda i, nb: (0, 0)),
            pl.BlockSpec((B, NCAND), lambda i, nb: (0, 0)),
            pl.BlockSpec((B, 1), lambda i, nb: (0, 0)),
        ],
        out_specs=[
            pl.BlockSpec((1, 1), lambda i, nb: (0, 0)),
            pl.BlockSpec((B, 1), lambda i, nb: (0, 0)),
        ],
        scratch_shapes=[pltpu.VMEM((B, D), jnp.float32)],
    )
    return pl.pallas_call(
        _tc_body,
        grid_spec=grid_spec,
        out_shape=[
            jax.ShapeDtypeStruct((1, 1), jnp.float32),
            jax.ShapeDtypeStruct((B, 1), jnp.int32),
        ],
    )(nb_arr, tok, w_enc, b_enc2, lo, hi, iw, wg, bgr, tgt2)


def kernel(context_ids, context_spans, sense_ids, target_ids, emb_table,
           W_enc, b_enc, sense_W, sense_b):
    context_ids = context_ids.astype(jnp.int32)
    context_spans = context_spans.astype(jnp.int32)
    sense_ids = sense_ids.astype(jnp.int32)
    target_ids = target_ids.astype(jnp.int32)

    start = context_spans[:, 0]
    end = jnp.maximum(context_spans[:, 1], start + 1)
    w = (end - start).astype(jnp.int32)
    cum = jnp.concatenate([jnp.zeros((1,), jnp.int32),
                           jnp.cumsum(w, dtype=jnp.int32)])
    t_total = cum[B]
    tp = ((t_total + 511) // 512) * 512
    cw = tp // NW
    nb = tp // BLK

    t = jnp.arange(TPAD, dtype=jnp.int32)
    ge_hi = (t[None, :] >= cum[1:, None]).astype(jnp.int32)
    b_of_t = jnp.minimum(jnp.sum(ge_hi, axis=0), B - 1)
    onehot = (b_of_t[None, :] == jnp.arange(B, dtype=jnp.int32)[:, None])
    start_sel = jnp.sum(jnp.where(onehot, start[:, None], 0), axis=0)
    cum_sel = jnp.sum(jnp.where(onehot, cum[:B, None], 0), axis=0)
    pos = start_sel + (t - cum_sel) + b_of_t * S
    pos = jnp.where(t < t_total, pos, 0)

    meta = jnp.full((16,), cw, dtype=jnp.int32)
    ctx_flat = context_ids.reshape(-1)
    sids_flat = sense_ids.reshape(-1)

    n_senses = sense_b.shape[0]
    pad_b = (-n_senses) % 128
    sb_rows = jnp.pad(sense_b, (0, pad_b)).reshape(-1, 128)
    tok, wg, bg = _sc_gather(ctx_flat, pos, meta, sids_flat,
                             emb_table, sense_W, sb_rows)

    nb_arr = jnp.reshape(nb, (1,)).astype(jnp.int32)
    lo = cum[:B].reshape(B, 1)
    hi = cum[1:].reshape(B, 1)
    iw = (1.0 / w.astype(jnp.float32)).reshape(B, 1)
    loss2, corr2 = _tc_forward(nb_arr, tok, W_enc, b_enc.reshape(1, D),
                               lo, hi, iw, wg, bg.reshape(B, NCAND),
                               target_ids.reshape(B, 1))
    return loss2[0, 0], corr2[:, 0].astype(jnp.bool_)
